# baseline (speedup 1.0000x reference)
import functools

import jax
import jax.numpy as jnp
from jax.experimental import pallas as pl
from jax.experimental.pallas import tpu as pltpu

_SEG = 784
_LCM = 6272
_CPR = 8


def _se_kernel(x_ref, st_ref, w1_ref, b1_ref, w2_ref, b2_ref, o_ref, *,
               inv_hw, nb, c, rows):
    st = st_ref[...]
    psum = jax.lax.dot_general(
        x_ref[...], st, (((2,), (1,)), ((), ())),
        preferred_element_type=jnp.float32)
    pooled = psum.reshape(nb, c) * inv_hw
    h = jnp.maximum(
        jax.lax.dot_general(pooled, w1_ref[...], (((1,), (1,)), ((), ())),
                            preferred_element_type=jnp.float32)
        + b1_ref[...], 0.0)
    g = jax.nn.sigmoid(
        jax.lax.dot_general(h, w2_ref[...], (((1,), (1,)), ((), ())),
                            preferred_element_type=jnp.float32)
        + b2_ref[...])
    gmap = jax.lax.dot_general(
        g.reshape(nb, rows, _CPR), st, (((2,), (0,)), ((), ())),
        preferred_element_type=jnp.float32)
    o_ref[...] = (x_ref[...] * gmap).astype(o_ref.dtype)


def _pick_images_per_block(n, bytes_per_image, budget):
    best = 1
    for d in range(1, n + 1):
        if n % d == 0 and d * bytes_per_image <= budget:
            best = d
    return best


def kernel(x_nchw, w1, b1, w2, b2):
    N, C, H, W = x_nchw.shape
    Cr = w1.shape[0]
    HW = H * W
    dtype = x_nchw.dtype

    if HW != _SEG or C % _CPR:
        return _kernel_padded(x_nchw, w1, b1, w2, b2)

    rows = C // _CPR
    x3 = x_nchw.reshape(N, rows, _LCM)
    b1r = b1.reshape(1, Cr)
    b2r = b2.reshape(1, C)
    inv_hw = 1.0 / float(HW)

    lane = jax.lax.broadcasted_iota(jnp.int32, (_CPR, _LCM), 1)
    ch = jax.lax.broadcasted_iota(jnp.int32, (_CPR, _LCM), 0)
    st = (lane // _SEG == ch).astype(jnp.float32)

    bytes_per_image = rows * _LCM * dtype.itemsize
    nb = _pick_images_per_block(N, bytes_per_image, budget=8 << 20)
    grid = (N // nb,)

    out3 = pl.pallas_call(
        functools.partial(_se_kernel, inv_hw=inv_hw, nb=nb, c=C, rows=rows),
        out_shape=jax.ShapeDtypeStruct((N, rows, _LCM), dtype),
        grid=grid,
        in_specs=[
            pl.BlockSpec((nb, rows, _LCM), lambda i: (i, 0, 0)),
            pl.BlockSpec((_CPR, _LCM), lambda i: (0, 0)),
            pl.BlockSpec((Cr, C), lambda i: (0, 0)),
            pl.BlockSpec((1, Cr), lambda i: (0, 0)),
            pl.BlockSpec((C, Cr), lambda i: (0, 0)),
            pl.BlockSpec((1, C), lambda i: (0, 0)),
        ],
        out_specs=pl.BlockSpec((nb, rows, _LCM), lambda i: (i, 0, 0)),
        compiler_params=pltpu.CompilerParams(
            dimension_semantics=("parallel",),
            vmem_limit_bytes=48 << 20,
        ),
    )(x3, st, w1, b1r, w2, b2r)

    return out3.reshape(N, C, H, W)


def _se_kernel_nc(x_ref, w1_ref, b1_ref, w2_ref, b2_ref, o_ref, *, inv_hw):
    pooled = jnp.sum(x_ref[...], axis=-1, dtype=jnp.float32) * inv_hw
    h = jnp.maximum(
        jax.lax.dot_general(pooled, w1_ref[...], (((1,), (1,)), ((), ())),
                            preferred_element_type=jnp.float32)
        + b1_ref[...], 0.0)
    g = jax.nn.sigmoid(
        jax.lax.dot_general(h, w2_ref[...], (((1,), (1,)), ((), ())),
                            preferred_element_type=jnp.float32)
        + b2_ref[...])
    o_ref[...] = (x_ref[...] * g[:, :, None]).astype(o_ref.dtype)


def _kernel_padded(x_nchw, w1, b1, w2, b2):
    N, C, H, W = x_nchw.shape
    Cr = w1.shape[0]
    HW = H * W
    dtype = x_nchw.dtype
    x3 = x_nchw.reshape(N, C, HW)
    b1r = b1.reshape(1, Cr)
    b2r = b2.reshape(1, C)
    inv_hw = 1.0 / float(HW)
    lanes = ((HW + 127) // 128) * 128
    bytes_per_image = C * lanes * dtype.itemsize
    nb = _pick_images_per_block(N, bytes_per_image, budget=4 << 20)
    out3 = pl.pallas_call(
        functools.partial(_se_kernel_nc, inv_hw=inv_hw),
        out_shape=jax.ShapeDtypeStruct((N, C, HW), dtype),
        grid=(N // nb,),
        in_specs=[
            pl.BlockSpec((nb, C, HW), lambda i: (i, 0, 0)),
            pl.BlockSpec((Cr, C), lambda i: (0, 0)),
            pl.BlockSpec((1, Cr), lambda i: (0, 0)),
            pl.BlockSpec((C, Cr), lambda i: (0, 0)),
            pl.BlockSpec((1, C), lambda i: (0, 0)),
        ],
        out_specs=pl.BlockSpec((nb, C, HW), lambda i: (i, 0, 0)),
        compiler_params=pltpu.CompilerParams(
            dimension_semantics=("parallel",),
            vmem_limit_bytes=48 << 20,
        ),
    )(x3, w1, b1r, w2, b2r)
    return out3.reshape(N, C, H, W)


# speedup vs baseline: 3.3987x; 3.3987x over previous
import functools

import jax
import jax.numpy as jnp
from jax.experimental import pallas as pl
from jax.experimental.pallas import tpu as pltpu


def _se_kernel(x_ref, w1_ref, b1_ref, w2_ref, b2_ref, o_ref, *, inv_hw):
    pooled = jnp.sum(x_ref[...], axis=-1, dtype=jnp.float32) * inv_hw
    h = jnp.maximum(
        jax.lax.dot_general(pooled, w1_ref[...],
                            (((1,), (1,)), ((), ())),
                            preferred_element_type=jnp.float32)
        + b1_ref[...], 0.0)
    g = jax.nn.sigmoid(
        jax.lax.dot_general(h, w2_ref[...],
                            (((1,), (1,)), ((), ())),
                            preferred_element_type=jnp.float32)
        + b2_ref[...])
    o_ref[...] = (x_ref[...] * g[:, :, None]).astype(o_ref.dtype)


def _pick_images_per_block(n, bytes_per_image, budget):
    best = 1
    for d in range(1, n + 1):
        if n % d == 0 and d * bytes_per_image <= budget:
            best = d
    return best


def kernel(x_nchw, w1, b1, w2, b2):
    N, C, H, W = x_nchw.shape
    Cr = w1.shape[0]
    HW = H * W
    dtype = x_nchw.dtype

    x3 = x_nchw.reshape(N, C, HW)
    b1r = b1.reshape(1, Cr)
    b2r = b2.reshape(1, C)
    inv_hw = 1.0 / float(HW)

    lanes = ((HW + 127) // 128) * 128
    bytes_per_image = C * lanes * dtype.itemsize
    nb = _pick_images_per_block(N, bytes_per_image, budget=8 << 20)
    grid = (N // nb,)

    out3 = pl.pallas_call(
        functools.partial(_se_kernel, inv_hw=inv_hw),
        out_shape=jax.ShapeDtypeStruct((N, C, HW), dtype),
        grid=grid,
        in_specs=[
            pl.BlockSpec((nb, C, HW), lambda i: (i, 0, 0)),
            pl.BlockSpec((Cr, C), lambda i: (0, 0)),
            pl.BlockSpec((1, Cr), lambda i: (0, 0)),
            pl.BlockSpec((C, Cr), lambda i: (0, 0)),
            pl.BlockSpec((1, C), lambda i: (0, 0)),
        ],
        out_specs=pl.BlockSpec((nb, C, HW), lambda i: (i, 0, 0)),
        compiler_params=pltpu.CompilerParams(
            dimension_semantics=("parallel",),
            vmem_limit_bytes=48 << 20,
        ),
    )(x3, w1, b1r, w2, b2r)

    return out3.reshape(N, C, H, W)
